# baseline (speedup 1.0000x reference)
import jax
import jax.numpy as jnp
from jax import lax
from jax.experimental import pallas as pl
from jax.experimental.pallas import tpu as pltpu
from jax.experimental.pallas import tpu_sc as plsc

_SEQ = 4096
_HID = 4096
_SLOTS = 64
_K = 8
_LANES = 16
_NTILES = 16
_PER_TILE = _SEQ // _NTILES
_NCHUNK = _PER_TILE // _LANES


def _norms_body(x_ref, o_ref):
    x = x_ref[...]
    o_ref[...] = jnp.sum(x * x, axis=1)[None, None, :]


def _tc_norms(pred2d):
    nblk = 16
    rows = _SEQ // nblk
    return pl.pallas_call(
        _norms_body,
        grid=(nblk,),
        in_specs=[pl.BlockSpec((rows, _HID), lambda i: (i, 0))],
        out_specs=pl.BlockSpec((1, 1, rows), lambda i: (i, 0, 0)),
        out_shape=jax.ShapeDtypeStruct((nblk, 1, rows), jnp.float32),
    )(pred2d)


def _merge_sorted(bk, bi, ck_s, ci_s):
    ck_r = lax.rev(ck_s, (0,))
    ci_r = lax.rev(ci_s, (0,))
    keep = bk >= ck_r
    mk = jnp.where(keep, bk, ck_r)
    mi = jnp.where(keep, bi, ci_r)
    nk, ni = plsc.sort_key_val(mk, mi, descending=True)
    return nk, ni


def _sc_body(norms_hbm, pred_hbm, states_hbm, strength_hbm,
             out_states_hbm, out_strength_hbm,
             norms_v, kv, iv, kvf, ivf, idx_v, rows_v, str_v, bank_v,
             sh_k, sh_i, sem):
    c = lax.axis_index("c")
    s = lax.axis_index("s")

    @pl.when(c == 0)
    def _topk():
        base = pl.multiple_of(s * _PER_TILE, _PER_TILE)
        pltpu.sync_copy(norms_hbm.at[pl.ds(base, _PER_TILE)], norms_v)
        lane = lax.iota(jnp.int32, _LANES)

        def local_merge(j, carry):
            bk, bi = carry
            ck = norms_v[pl.ds(j * _LANES, _LANES)]
            ci = lane + (base + j * _LANES)
            ck_s, ci_s = plsc.sort_key_val(ck, ci, descending=True)
            return _merge_sorted(bk, bi, ck_s, ci_s)

        bk0 = jnp.full((_LANES,), -jnp.inf, jnp.float32)
        bi0 = jnp.zeros((_LANES,), jnp.int32)
        bk, bi = lax.fori_loop(0, _NCHUNK, local_merge, (bk0, bi0))
        kv[...] = bk
        iv[...] = bi
        pltpu.sync_copy(kv, sh_k.at[pl.ds(s * _LANES, _LANES)])
        pltpu.sync_copy(iv, sh_i.at[pl.ds(s * _LANES, _LANES)])
        plsc.subcore_barrier()

        @pl.when(s == 0)
        def _final():
            pltpu.sync_copy(sh_k, kvf)
            pltpu.sync_copy(sh_i, ivf)

            def final_merge(j, carry):
                bk2, bi2 = carry
                ck_s = kvf[pl.ds(j * _LANES, _LANES)]
                ci_s = ivf[pl.ds(j * _LANES, _LANES)]
                return _merge_sorted(bk2, bi2, ck_s, ci_s)

            fk, fi = lax.fori_loop(0, _NTILES, final_merge, (bk0, bi0))
            idx_v[...] = fi
            pltpu.async_copy(pred_hbm.at[idx_v], rows_v, sem).wait()
            pltpu.sync_copy(rows_v.at[pl.ds(0, _K)],
                            out_states_hbm.at[pl.ds(0, _K)])
            pltpu.sync_copy(strength_hbm, str_v)
            s0 = str_v[pl.ds(0, _LANES)]
            str_v[pl.ds(0, _LANES)] = jnp.where(lane < _K, jnp.float32(1.0), s0)
            pltpu.sync_copy(str_v, out_strength_hbm)

    @pl.when((c == 1) & (s < 14))
    def _copy_bank():
        r0 = _K + s * 4
        pltpu.sync_copy(states_hbm.at[pl.ds(r0, 4)], bank_v)
        pltpu.sync_copy(bank_v, out_states_hbm.at[pl.ds(r0, 4)])


def kernel(predictions, mem_states, mem_strength, top_k):
    del top_k
    pred2d = predictions.reshape(2 * _SEQ, _HID)
    norms = _tc_norms(pred2d).reshape(_SEQ)
    sc = pl.kernel(
        _sc_body,
        mesh=plsc.VectorSubcoreMesh(core_axis_name="c", subcore_axis_name="s"),
        compiler_params=pltpu.CompilerParams(needs_layout_passes=False),
        out_type=[
            jax.ShapeDtypeStruct((_SLOTS, _HID), jnp.float32),
            jax.ShapeDtypeStruct((_SLOTS,), jnp.float32),
        ],
        scratch_types=[
            pltpu.VMEM((_PER_TILE,), jnp.float32),
            pltpu.VMEM((_LANES,), jnp.float32),
            pltpu.VMEM((_LANES,), jnp.int32),
            pltpu.VMEM((_SEQ // _NCHUNK,), jnp.float32),
            pltpu.VMEM((_SEQ // _NCHUNK,), jnp.int32),
            pltpu.VMEM((_LANES,), jnp.int32),
            pltpu.VMEM((_LANES, _HID), jnp.float32),
            pltpu.VMEM((_SLOTS,), jnp.float32),
            pltpu.VMEM((4, _HID), jnp.float32),
            pltpu.VMEM_SHARED((_NTILES * _LANES,), jnp.float32),
            pltpu.VMEM_SHARED((_NTILES * _LANES,), jnp.int32),
            pltpu.SemaphoreType.DMA,
        ],
    )
    new_states, new_strength = sc(norms, pred2d, mem_states, mem_strength)
    return new_states, new_strength


# speedup vs baseline: 1.0176x; 1.0176x over previous
import jax
import jax.numpy as jnp
from jax import lax
from jax.experimental import pallas as pl
from jax.experimental.pallas import tpu as pltpu
from jax.experimental.pallas import tpu_sc as plsc

_SEQ = 4096
_HID = 4096
_SLOTS = 64
_K = 8
_LANES = 16
_NTILES = 16
_PER_TILE = _SEQ // _NTILES
_NCHUNK = _PER_TILE // _LANES


def _norms_body(x_ref, o_ref):
    x = x_ref[...]
    o_ref[...] = jnp.sum(x * x, axis=1)[None, None, :]


def _tc_norms(pred2d):
    nblk = 4
    rows = _SEQ // nblk
    return pl.pallas_call(
        _norms_body,
        grid=(nblk,),
        in_specs=[pl.BlockSpec((rows, _HID), lambda i: (i, 0))],
        out_specs=pl.BlockSpec((1, 1, rows), lambda i: (i, 0, 0)),
        out_shape=jax.ShapeDtypeStruct((nblk, 1, rows), jnp.float32),
    )(pred2d)


def _merge_sorted(bk, bi, ck_s, ci_s):
    ck_r = lax.rev(ck_s, (0,))
    ci_r = lax.rev(ci_s, (0,))
    keep = bk >= ck_r
    mk = jnp.where(keep, bk, ck_r)
    mi = jnp.where(keep, bi, ci_r)
    nk, ni = plsc.sort_key_val(mk, mi, descending=True)
    return nk, ni


def _sc_body(norms_hbm, pred_hbm, states_hbm, strength_hbm,
             out_states_hbm, out_strength_hbm,
             norms_v, kv, iv, kvf, ivf, idx_v, rows_v, str_v, bank_v,
             sh_k, sh_i, sem):
    c = lax.axis_index("c")
    s = lax.axis_index("s")

    @pl.when(c == 0)
    def _topk():
        base = pl.multiple_of(s * _PER_TILE, _PER_TILE)
        pltpu.sync_copy(norms_hbm.at[pl.ds(base, _PER_TILE)], norms_v)
        lane = lax.iota(jnp.int32, _LANES)

        def local_merge(j, carry):
            bk, bi = carry
            ck = norms_v[pl.ds(j * _LANES, _LANES)]
            ci = lane + (base + j * _LANES)
            ck_s, ci_s = plsc.sort_key_val(ck, ci, descending=True)
            return _merge_sorted(bk, bi, ck_s, ci_s)

        bk0 = jnp.full((_LANES,), -jnp.inf, jnp.float32)
        bi0 = jnp.zeros((_LANES,), jnp.int32)
        bk, bi = lax.fori_loop(0, _NCHUNK, local_merge, (bk0, bi0))
        kv[...] = bk
        iv[...] = bi
        pltpu.sync_copy(kv, sh_k.at[pl.ds(s * _LANES, _LANES)])
        pltpu.sync_copy(iv, sh_i.at[pl.ds(s * _LANES, _LANES)])
        plsc.subcore_barrier()

        @pl.when(s == 0)
        def _final():
            pltpu.sync_copy(sh_k, kvf)
            pltpu.sync_copy(sh_i, ivf)

            def final_merge(j, carry):
                bk2, bi2 = carry
                ck_s = kvf[pl.ds(j * _LANES, _LANES)]
                ci_s = ivf[pl.ds(j * _LANES, _LANES)]
                return _merge_sorted(bk2, bi2, ck_s, ci_s)

            fk, fi = lax.fori_loop(0, _NTILES, final_merge, (bk0, bi0))
            idx_v[...] = fi
            pltpu.async_copy(pred_hbm.at[idx_v], rows_v, sem).wait()
            pltpu.sync_copy(rows_v.at[pl.ds(0, _K)],
                            out_states_hbm.at[pl.ds(0, _K)])
            pltpu.sync_copy(strength_hbm, str_v)
            s0 = str_v[pl.ds(0, _LANES)]
            str_v[pl.ds(0, _LANES)] = jnp.where(lane < _K, jnp.float32(1.0), s0)
            pltpu.sync_copy(str_v, out_strength_hbm)

    @pl.when((c == 1) & (s < 14))
    def _copy_bank():
        r0 = _K + s * 4
        pltpu.sync_copy(states_hbm.at[pl.ds(r0, 4)], bank_v)
        pltpu.sync_copy(bank_v, out_states_hbm.at[pl.ds(r0, 4)])


def kernel(predictions, mem_states, mem_strength, top_k):
    del top_k
    pred2d = predictions.reshape(2 * _SEQ, _HID)
    norms = _tc_norms(pred2d).reshape(_SEQ)
    sc = pl.kernel(
        _sc_body,
        mesh=plsc.VectorSubcoreMesh(core_axis_name="c", subcore_axis_name="s"),
        compiler_params=pltpu.CompilerParams(needs_layout_passes=False),
        out_type=[
            jax.ShapeDtypeStruct((_SLOTS, _HID), jnp.float32),
            jax.ShapeDtypeStruct((_SLOTS,), jnp.float32),
        ],
        scratch_types=[
            pltpu.VMEM((_PER_TILE,), jnp.float32),
            pltpu.VMEM((_LANES,), jnp.float32),
            pltpu.VMEM((_LANES,), jnp.int32),
            pltpu.VMEM((_SEQ // _NCHUNK,), jnp.float32),
            pltpu.VMEM((_SEQ // _NCHUNK,), jnp.int32),
            pltpu.VMEM((_LANES,), jnp.int32),
            pltpu.VMEM((_LANES, _HID), jnp.float32),
            pltpu.VMEM((_SLOTS,), jnp.float32),
            pltpu.VMEM((4, _HID), jnp.float32),
            pltpu.VMEM_SHARED((_NTILES * _LANES,), jnp.float32),
            pltpu.VMEM_SHARED((_NTILES * _LANES,), jnp.int32),
            pltpu.SemaphoreType.DMA,
        ],
    )
    new_states, new_strength = sc(norms, pred2d, mem_states, mem_strength)
    return new_states, new_strength


# speedup vs baseline: 1.0463x; 1.0282x over previous
import jax
import jax.numpy as jnp
from jax import lax
from jax.experimental import pallas as pl
from jax.experimental.pallas import tpu as pltpu
from jax.experimental.pallas import tpu_sc as plsc

_SEQ = 4096
_HID = 4096
_SLOTS = 64
_K = 8
_LANES = 16
_NTILES = 16
_PER_TILE = _SEQ // _NTILES
_NCHUNK = _PER_TILE // _LANES


def _norms_body(a_ref, b_ref, o_ref):
    a = a_ref[...]
    b = b_ref[...]
    o_ref[...] = (jnp.sum(a * a, axis=1) + jnp.sum(b * b, axis=1))[None, None, :]


def _tc_norms(pred2d):
    nblk = 8
    rows = _SEQ // nblk
    half = _HID // 2
    return pl.pallas_call(
        _norms_body,
        grid=(nblk,),
        in_specs=[
            pl.BlockSpec((rows, half), lambda i: (i, 0)),
            pl.BlockSpec((rows, half), lambda i: (i, 1)),
        ],
        out_specs=pl.BlockSpec((1, 1, rows), lambda i: (i, 0, 0)),
        out_shape=jax.ShapeDtypeStruct((nblk, 1, rows), jnp.float32),
    )(pred2d, pred2d)


def _merge_sorted(bk, bi, ck_s, ci_s):
    ck_r = lax.rev(ck_s, (0,))
    ci_r = lax.rev(ci_s, (0,))
    keep = bk >= ck_r
    mk = jnp.where(keep, bk, ck_r)
    mi = jnp.where(keep, bi, ci_r)
    nk, ni = plsc.sort_key_val(mk, mi, descending=True)
    return nk, ni


def _sc_body(norms_hbm, pred_hbm, states_hbm, strength_hbm,
             out_states_hbm, out_strength_hbm,
             norms_v, kv, iv, kvf, ivf, idx_v, rows_v, str_v, bank_v,
             sh_k, sh_i, sem):
    c = lax.axis_index("c")
    s = lax.axis_index("s")

    @pl.when(c == 0)
    def _topk():
        base = pl.multiple_of(s * _PER_TILE, _PER_TILE)
        pltpu.sync_copy(norms_hbm.at[pl.ds(base, _PER_TILE)], norms_v)
        lane = lax.iota(jnp.int32, _LANES)

        def local_merge(j, carry):
            bk, bi = carry
            ck = norms_v[pl.ds(j * _LANES, _LANES)]
            ci = lane + (base + j * _LANES)
            ck_s, ci_s = plsc.sort_key_val(ck, ci, descending=True)
            return _merge_sorted(bk, bi, ck_s, ci_s)

        bk0 = jnp.full((_LANES,), -jnp.inf, jnp.float32)
        bi0 = jnp.zeros((_LANES,), jnp.int32)
        bk, bi = lax.fori_loop(0, _NCHUNK, local_merge, (bk0, bi0))
        kv[...] = bk
        iv[...] = bi
        pltpu.sync_copy(kv, sh_k.at[pl.ds(s * _LANES, _LANES)])
        pltpu.sync_copy(iv, sh_i.at[pl.ds(s * _LANES, _LANES)])
        plsc.subcore_barrier()

        @pl.when(s == 0)
        def _final():
            pltpu.sync_copy(sh_k, kvf)
            pltpu.sync_copy(sh_i, ivf)

            def final_merge(j, carry):
                bk2, bi2 = carry
                ck_s = kvf[pl.ds(j * _LANES, _LANES)]
                ci_s = ivf[pl.ds(j * _LANES, _LANES)]
                return _merge_sorted(bk2, bi2, ck_s, ci_s)

            fk, fi = lax.fori_loop(0, _NTILES, final_merge, (bk0, bi0))
            idx_v[...] = fi
            pltpu.async_copy(pred_hbm.at[idx_v], rows_v, sem).wait()
            pltpu.sync_copy(rows_v.at[pl.ds(0, _K)],
                            out_states_hbm.at[pl.ds(0, _K)])
            pltpu.sync_copy(strength_hbm, str_v)
            s0 = str_v[pl.ds(0, _LANES)]
            str_v[pl.ds(0, _LANES)] = jnp.where(lane < _K, jnp.float32(1.0), s0)
            pltpu.sync_copy(str_v, out_strength_hbm)

    @pl.when((c == 1) & (s < 14))
    def _copy_bank():
        r0 = _K + s * 4
        pltpu.sync_copy(states_hbm.at[pl.ds(r0, 4)], bank_v)
        pltpu.sync_copy(bank_v, out_states_hbm.at[pl.ds(r0, 4)])


def kernel(predictions, mem_states, mem_strength, top_k):
    del top_k
    pred2d = predictions.reshape(2 * _SEQ, _HID)
    norms = _tc_norms(pred2d).reshape(_SEQ)
    sc = pl.kernel(
        _sc_body,
        mesh=plsc.VectorSubcoreMesh(core_axis_name="c", subcore_axis_name="s"),
        compiler_params=pltpu.CompilerParams(needs_layout_passes=False),
        out_type=[
            jax.ShapeDtypeStruct((_SLOTS, _HID), jnp.float32),
            jax.ShapeDtypeStruct((_SLOTS,), jnp.float32),
        ],
        scratch_types=[
            pltpu.VMEM((_PER_TILE,), jnp.float32),
            pltpu.VMEM((_LANES,), jnp.float32),
            pltpu.VMEM((_LANES,), jnp.int32),
            pltpu.VMEM((_SEQ // _NCHUNK,), jnp.float32),
            pltpu.VMEM((_SEQ // _NCHUNK,), jnp.int32),
            pltpu.VMEM((_LANES,), jnp.int32),
            pltpu.VMEM((_LANES, _HID), jnp.float32),
            pltpu.VMEM((_SLOTS,), jnp.float32),
            pltpu.VMEM((4, _HID), jnp.float32),
            pltpu.VMEM_SHARED((_NTILES * _LANES,), jnp.float32),
            pltpu.VMEM_SHARED((_NTILES * _LANES,), jnp.int32),
            pltpu.SemaphoreType.DMA,
        ],
    )
    new_states, new_strength = sc(norms, pred2d, mem_states, mem_strength)
    return new_states, new_strength
